# Initial kernel scaffold; baseline (speedup 1.0000x reference)
import jax
import jax.numpy as jnp
from jax.experimental import pallas as pl

ROW_BLOCK = 1000


def _mm_body(x_ref, w_ref, o_ref):
    o_ref[...] = jnp.dot(x_ref[...], w_ref[...], preferred_element_type=jnp.float32)


def _mm(x, w):
    n, fi = x.shape
    fo = w.shape[1]
    grid = (n // ROW_BLOCK,)
    return pl.pallas_call(
        _mm_body,
        grid=grid,
        in_specs=[
            pl.BlockSpec((ROW_BLOCK, fi), lambda i: (i, 0)),
            pl.BlockSpec((fi, fo), lambda i: (0, 0)),
        ],
        out_specs=pl.BlockSpec((ROW_BLOCK, fo), lambda i: (i, 0)),
        out_shape=jax.ShapeDtypeStruct((n, fo), jnp.float32),
    )(x, w)


def kernel(x, edge_index, We0, be0, We1, be1, We2, be2, Wd0, bd0, Wd1, bd1, Wd2, bd2):
    n = x.shape[0]
    src, dst = edge_index[0], edge_index[1]
    cnt = jnp.zeros((n,), jnp.float32).at[dst].add(1.0)
    dis_c = jax.lax.rsqrt(cnt + 1.0)
    dis_d = jnp.where(cnt > 0, jax.lax.rsqrt(jnp.maximum(cnt, 1e-12)), 0.0)

    def agg(h2):
        return jnp.zeros_like(h2).at[dst].add(h2[src])

    z = x
    for W, b in ((We0, be0), (We1, be1), (We2, be2)):
        h2 = dis_c[:, None] * _mm(z, W)
        z = jax.nn.relu(dis_c[:, None] * (agg(h2) + h2) + b)
    enc = z
    h = enc
    for i, (W, b) in enumerate(((Wd0, bd0), (Wd1, bd1), (Wd2, bd2))):
        t = 2.0 * h - dis_d[:, None] * agg(dis_d[:, None] * h)
        h = _mm(t, W) + b
        if i < 2:
            h = jax.nn.relu(h)
    return (h, enc)


# speedup vs baseline: 2.1936x; 2.1936x over previous
import jax
import jax.numpy as jnp
from jax.experimental import pallas as pl

ROW_BLOCK = 1000


def _mm_body(x_ref, w_ref, o_ref):
    o_ref[...] = jnp.dot(x_ref[...], w_ref[...], preferred_element_type=jnp.float32)


def _mm(x, w):
    n, fi = x.shape
    fo = w.shape[1]
    grid = (n // ROW_BLOCK,)
    return pl.pallas_call(
        _mm_body,
        grid=grid,
        in_specs=[
            pl.BlockSpec((ROW_BLOCK, fi), lambda i: (i, 0)),
            pl.BlockSpec((fi, fo), lambda i: (0, 0)),
        ],
        out_specs=pl.BlockSpec((ROW_BLOCK, fo), lambda i: (i, 0)),
        out_shape=jax.ShapeDtypeStruct((n, fo), jnp.float32),
    )(x, w)


def kernel(x, edge_index, We0, be0, We1, be1, We2, be2, Wd0, bd0, Wd1, bd1, Wd2, bd2):
    n = x.shape[0]
    src, dst = edge_index[0], edge_index[1]
    cnt = jnp.zeros((n,), jnp.float32).at[dst].add(1.0)
    dis_c = jax.lax.rsqrt(cnt + 1.0)
    dis_d = jnp.where(cnt > 0, jax.lax.rsqrt(jnp.maximum(cnt, 1e-12)), 0.0)

    def agg(h2):
        return jnp.zeros_like(h2).at[dst].add(h2[src])

    z = x
    for W, b in ((We0, be0), (We1, be1), (We2, be2)):
        h2 = dis_c[:, None] * _mm(z, W)
        z = jax.nn.relu(dis_c[:, None] * (agg(h2) + h2) + b)
    enc = z
    h = enc
    for i, (W, b) in enumerate(((Wd0, bd0), (Wd1, bd1), (Wd2, bd2))):
        g = _mm(h, W)
        g2 = dis_d[:, None] * g
        h = 2.0 * g - dis_d[:, None] * agg(g2) + b
        if i < 2:
            h = jax.nn.relu(h)
    return (h, enc)


# speedup vs baseline: 9.5244x; 4.3418x over previous
import functools

import jax
import jax.numpy as jnp
from jax import lax
from jax.experimental import pallas as pl
from jax.experimental.pallas import tpu as pltpu
from jax.experimental.pallas import tpu_sc as plsc

N = 10000
NP = 10240
E = 320000
NW = 32
K = 128
C = 80
EP = NW * C * K
R = 1024
NS = 16
ROWS_PER_SUB = NP // NS
ZB = 128


def _mesh():
    return plsc.VectorSubcoreMesh(core_axis_name="c", subcore_axis_name="s")


CW = 16


def _cnt_body(dstc, ones, zros, out, idx_d, ones_v, zbuf, cnt_sp):
    c = lax.axis_index("c")
    s = lax.axis_index("s")
    w = c * NS + s
    pltpu.sync_copy(dstc.at[w], idx_d)
    pltpu.sync_copy(ones, ones_v)
    pltpu.sync_copy(zros, zbuf)
    for i in range(ROWS_PER_SUB // ZB):
        pltpu.sync_copy(zbuf, cnt_sp.at[pl.ds(s * ROWS_PER_SUB + i * ZB, ZB)])
    plsc.subcore_barrier()

    def step(j, _):
        pltpu.sync_copy(ones_v, cnt_sp.at[idx_d.at[j]], add=True)
        return 0

    lax.fori_loop(0, C, step, 0)
    plsc.subcore_barrier()
    pltpu.sync_copy(cnt_sp.at[pl.ds(s * ROWS_PER_SUB, ROWS_PER_SUB)],
                    out.at[c, pl.ds(s * ROWS_PER_SUB, ROWS_PER_SUB)])


def _sc_cnt(dstc):
    return pl.kernel(
        _cnt_body,
        out_type=jax.ShapeDtypeStruct((2, NP, CW), jnp.float32),
        mesh=_mesh(),
        scratch_types=[
            pltpu.VMEM((C, K), jnp.int32),
            pltpu.VMEM((K, CW), jnp.float32),
            pltpu.VMEM((ZB, CW), jnp.float32),
            pltpu.VMEM_SHARED((NP, CW), jnp.float32),
        ],
    )(dstc, jnp.ones((K, CW), jnp.float32), jnp.zeros((ZB, CW), jnp.float32))


def _agg_body(h_hbm, srcc, dstc, zros, out, idx_s, idx_d, gbuf, zbuf, out_sp,
              sem0, sem1):
    c = lax.axis_index("c")
    s = lax.axis_index("s")
    w = c * NS + s
    pltpu.sync_copy(srcc.at[w], idx_s)
    pltpu.sync_copy(dstc.at[w], idx_d)
    pltpu.sync_copy(zros, zbuf)
    for i in range(ROWS_PER_SUB // ZB):
        pltpu.sync_copy(zbuf, out_sp.at[pl.ds(s * ROWS_PER_SUB + i * ZB, ZB)])
    plsc.subcore_barrier()

    pltpu.async_copy(h_hbm.at[idx_s.at[0]], gbuf.at[0], sem0)

    def step(t, _):
        j0 = 2 * t
        j1 = j0 + 1
        d1 = pltpu.async_copy(h_hbm.at[idx_s.at[j1]], gbuf.at[1], sem1)
        pltpu.make_async_copy(h_hbm.at[idx_s.at[j0]], gbuf.at[0], sem0).wait()
        pltpu.sync_copy(gbuf.at[0], out_sp.at[idx_d.at[j0]], add=True)

        @pl.when(t + 1 < C // 2)
        def _():
            pltpu.async_copy(h_hbm.at[idx_s.at[j0 + 2]], gbuf.at[0], sem0)

        d1.wait()
        pltpu.sync_copy(gbuf.at[1], out_sp.at[idx_d.at[j1]], add=True)
        return 0

    lax.fori_loop(0, C // 2, step, 0)
    plsc.subcore_barrier()
    pltpu.sync_copy(out_sp.at[pl.ds(s * ROWS_PER_SUB, ROWS_PER_SUB)],
                    out.at[c, pl.ds(s * ROWS_PER_SUB, ROWS_PER_SUB)])


def _sc_agg(h, srcc, dstc, f):
    return pl.kernel(
        _agg_body,
        out_type=jax.ShapeDtypeStruct((2, NP, f), jnp.float32),
        mesh=_mesh(),
        compiler_params=pltpu.CompilerParams(use_tc_tiling_on_sc=False),
        scratch_types=[
            pltpu.VMEM((C, K), jnp.int32),
            pltpu.VMEM((C, K), jnp.int32),
            pltpu.VMEM((2, K, f), jnp.float32),
            pltpu.VMEM((ZB, f), jnp.float32),
            pltpu.VMEM_SHARED((NP, f), jnp.float32),
            pltpu.SemaphoreType.DMA,
            pltpu.SemaphoreType.DMA,
        ],
    )(h, srcc, dstc, jnp.zeros((ZB, f), jnp.float32))


def _grid_spec(in_blocks, out_blocks):
    return dict(
        grid=(NP // R,),
        in_specs=[pl.BlockSpec(b, ix) for b, ix in in_blocks],
        out_specs=[pl.BlockSpec(b, ix) for b, ix in out_blocks],
    )


def _row(i):
    return (i, 0)


def _full(i):
    return (0,)


def _pre_body(cntp_ref, x_ref, w_ref, h2_ref, disc_ref, disd_ref):
    cp = cntp_ref[...]
    cnt = cp[0, :, 0:1] + cp[1, :, 0:1]
    disc = jax.lax.rsqrt(cnt + 1.0)
    disd = jnp.where(cnt > 0.0,
                     jax.lax.rsqrt(jnp.maximum(cnt, 1e-12)), 0.0)
    disc_ref[...] = disc
    disd_ref[...] = disd
    h2_ref[...] = disc * jnp.dot(x_ref[...], w_ref[...],
                                 preferred_element_type=jnp.float32)


def _tc_pre(cntp, xp, w0):
    fi, fo = w0.shape
    spec = _grid_spec(
        [((2, R, CW), lambda i: (0, i, 0)), ((R, fi), _row),
         ((fi, fo), lambda i: (0, 0))],
        [((R, fo), _row), ((R, 1), _row), ((R, 1), _row)],
    )
    return pl.pallas_call(
        _pre_body,
        out_shape=[
            jax.ShapeDtypeStruct((NP, fo), jnp.float32),
            jax.ShapeDtypeStruct((NP, 1), jnp.float32),
            jax.ShapeDtypeStruct((NP, 1), jnp.float32),
        ],
        **spec,
    )(cntp, xp, w0)


def _mid_body(pp_ref, h2_ref, b_ref, disc_ref, w_ref, out_ref):
    disc = disc_ref[...]
    z = jnp.maximum(disc * (pp_ref[0] + pp_ref[1] + h2_ref[...]) + b_ref[...],
                    0.0)
    out_ref[...] = disc * jnp.dot(z, w_ref[...],
                                  preferred_element_type=jnp.float32)


def _tc_mid(pp, h2, b, disc, wn):
    fi, fo = wn.shape
    spec = _grid_spec(
        [((2, R, fi), lambda i: (0, i, 0)), ((R, fi), _row),
         ((fi,), _full), ((R, 1), _row), ((fi, fo), lambda i: (0, 0))],
        [((R, fo), _row)],
    )
    (out,) = pl.pallas_call(
        _mid_body,
        out_shape=[jax.ShapeDtypeStruct((NP, fo), jnp.float32)],
        **spec,
    )(pp, h2, b, disc, wn)
    return out


def _dec1_body(pp_ref, h2_ref, b_ref, disc_ref, disd_ref, w_ref,
               enc_ref, g_ref, g2_ref):
    disc = disc_ref[...]
    z = jnp.maximum(disc * (pp_ref[0] + pp_ref[1] + h2_ref[...]) + b_ref[...],
                    0.0)
    enc_ref[...] = z
    g = jnp.dot(z, w_ref[...], preferred_element_type=jnp.float32)
    g_ref[...] = g
    g2_ref[...] = disd_ref[...] * g


def _tc_dec1(pp, h2, b, disc, disd, wn):
    fi, fo = wn.shape
    spec = _grid_spec(
        [((2, R, fi), lambda i: (0, i, 0)), ((R, fi), _row),
         ((fi,), _full), ((R, 1), _row), ((R, 1), _row),
         ((fi, fo), lambda i: (0, 0))],
        [((R, fi), _row), ((R, fo), _row), ((R, fo), _row)],
    )
    return pl.pallas_call(
        _dec1_body,
        out_shape=[
            jax.ShapeDtypeStruct((NP, fi), jnp.float32),
            jax.ShapeDtypeStruct((NP, fo), jnp.float32),
            jax.ShapeDtypeStruct((NP, fo), jnp.float32),
        ],
        **spec,
    )(pp, h2, b, disc, disd, wn)


def _dec_body(qq_ref, g_ref, b_ref, disd_ref, w_ref, gn_ref, g2_ref):
    disd = disd_ref[...]
    h = jnp.maximum(2.0 * g_ref[...] - disd * (qq_ref[0] + qq_ref[1])
                    + b_ref[...], 0.0)
    gn = jnp.dot(h, w_ref[...], preferred_element_type=jnp.float32)
    gn_ref[...] = gn
    g2_ref[...] = disd * gn


def _tc_dec(qq, g, b, disd, wn):
    fi, fo = wn.shape
    spec = _grid_spec(
        [((2, R, fi), lambda i: (0, i, 0)), ((R, fi), _row),
         ((fi,), _full), ((R, 1), _row), ((fi, fo), lambda i: (0, 0))],
        [((R, fo), _row), ((R, fo), _row)],
    )
    return pl.pallas_call(
        _dec_body,
        out_shape=[
            jax.ShapeDtypeStruct((NP, fo), jnp.float32),
            jax.ShapeDtypeStruct((NP, fo), jnp.float32),
        ],
        **spec,
    )(qq, g, b, disd, wn)


def _dec_split_body(qq_ref, g_ref, b_ref, disd_ref, w_ref,
                    gn_ref, g2a_ref, g2b_ref):
    disd = disd_ref[...]
    h = jnp.maximum(2.0 * g_ref[...] - disd * (qq_ref[0] + qq_ref[1])
                    + b_ref[...], 0.0)
    gn = jnp.dot(h, w_ref[...], preferred_element_type=jnp.float32)
    gn_ref[...] = gn
    g2 = disd * gn
    g2a_ref[...] = g2[:, :64]
    g2b_ref[...] = g2[:, 64:]


def _tc_dec_split(qq, g, b, disd, wn):
    fi, fo = wn.shape
    spec = _grid_spec(
        [((2, R, fi), lambda i: (0, i, 0)), ((R, fi), _row),
         ((fi,), _full), ((R, 1), _row), ((fi, fo), lambda i: (0, 0))],
        [((R, fo), _row), ((R, fo // 2), _row), ((R, fo // 2), _row)],
    )
    return pl.pallas_call(
        _dec_split_body,
        out_shape=[
            jax.ShapeDtypeStruct((NP, fo), jnp.float32),
            jax.ShapeDtypeStruct((NP, fo // 2), jnp.float32),
            jax.ShapeDtypeStruct((NP, fo // 2), jnp.float32),
        ],
        **spec,
    )(qq, g, b, disd, wn)


def _fin_body(qqa_ref, qqb_ref, g_ref, b_ref, disd_ref, out_ref):
    disd = disd_ref[...]
    agg = jnp.concatenate(
        [qqa_ref[0] + qqa_ref[1], qqb_ref[0] + qqb_ref[1]], axis=1)
    out_ref[...] = 2.0 * g_ref[...] - disd * agg + b_ref[...]


def _tc_fin(qqa, qqb, g, b, disd):
    fo = g.shape[1]
    fh = fo // 2
    spec = _grid_spec(
        [((2, R, fh), lambda i: (0, i, 0)), ((2, R, fh), lambda i: (0, i, 0)),
         ((R, fo), _row), ((fo,), _full), ((R, 1), _row)],
        [((R, fo), _row)],
    )
    (out,) = pl.pallas_call(
        _fin_body,
        out_shape=[jax.ShapeDtypeStruct((NP, fo), jnp.float32)],
        **spec,
    )(qqa, qqb, g, b, disd)
    return out


def kernel(x, edge_index, We0, be0, We1, be1, We2, be2, Wd0, bd0, Wd1, bd1,
           Wd2, bd2):
    src, dst = edge_index[0], edge_index[1]
    pad_e = EP - E
    srcc = jnp.concatenate([src, jnp.zeros((pad_e,), jnp.int32)]).reshape(NW, C, K)
    dstc = jnp.concatenate([dst, jnp.full((pad_e,), N, jnp.int32)]).reshape(NW, C, K)
    xp = jnp.concatenate([x, jnp.zeros((NP - N, x.shape[1]), jnp.float32)])

    cntp = _sc_cnt(dstc)
    h2, disc, disd = _tc_pre(cntp, xp, We0)
    p = _sc_agg(h2, srcc, dstc, 96)
    h2 = _tc_mid(p, h2, be0, disc, We1)
    p = _sc_agg(h2, srcc, dstc, 64)
    h2 = _tc_mid(p, h2, be1, disc, We2)
    p = _sc_agg(h2, srcc, dstc, 32)
    enc, g, g2 = _tc_dec1(p, h2, be2, disc, disd, Wd0)
    q = _sc_agg(g2, srcc, dstc, 64)
    g, g2 = _tc_dec(q, g, bd0, disd, Wd1)
    q = _sc_agg(g2, srcc, dstc, 96)
    g, g2a, g2b = _tc_dec_split(q, g, bd1, disd, Wd2)
    qa = _sc_agg(g2a, srcc, dstc, 64)
    qb = _sc_agg(g2b, srcc, dstc, 64)
    x_hat = _tc_fin(qa, qb, g, bd2, disd)
    return (x_hat[:N], enc[:N])
